# baseline (speedup 1.0000x reference)
import jax
import jax.numpy as jnp
from jax.experimental import pallas as pl
from jax.experimental.pallas import tpu as pltpu

N = 10000
D = 128
C = 16
BN = 2000
GRID = N // BN

_SELU_ALPHA = 1.6732632423543772848170429916717
_SELU_SCALE = 1.0507009873554804934193349852946


def _dmon_kernel(x_ref, w_ref, b_ref, pooled_ref, assign_ref, s_ref):
    i = pl.program_id(0)

    x = x_ref[...]
    logits = jax.lax.dot_general(
        x, w_ref[...],
        (((1,), (0,)), ((), ())),
        precision=jax.lax.Precision.HIGHEST,
        preferred_element_type=jnp.float32,
    ) + b_ref[...]

    m = jnp.max(logits, axis=1, keepdims=True)
    e = jnp.exp(logits - m)
    a = e / jnp.sum(e, axis=1, keepdims=True)
    assign_ref[...] = a

    part = jax.lax.dot_general(
        a, x,
        (((0,), (0,)), ((), ())),
        precision=jax.lax.Precision.HIGHEST,
        preferred_element_type=jnp.float32,
    )
    part_s = jnp.sum(a, axis=0, keepdims=True)

    @pl.when(i == 0)
    def _init():
        pooled_ref[...] = part
        s_ref[...] = part_s

    @pl.when(i > 0)
    def _acc():
        pooled_ref[...] += part
        s_ref[...] += part_s

    @pl.when(i == GRID - 1)
    def _finalize():
        pooled = pooled_ref[...] / s_ref[...].reshape(C, 1)
        pooled_ref[...] = _SELU_SCALE * jnp.where(
            pooled > 0, pooled, _SELU_ALPHA * (jnp.exp(pooled) - 1.0)
        )


def kernel(features, edge_index, W, b):
    del edge_index
    b2 = b.reshape(1, C)
    features_pooled, assignments = pl.pallas_call(
        _dmon_kernel,
        grid=(GRID,),
        in_specs=[
            pl.BlockSpec((BN, D), lambda i: (i, 0)),
            pl.BlockSpec((D, C), lambda i: (0, 0)),
            pl.BlockSpec((1, C), lambda i: (0, 0)),
        ],
        out_specs=[
            pl.BlockSpec((C, D), lambda i: (0, 0)),
            pl.BlockSpec((BN, C), lambda i: (i, 0)),
        ],
        out_shape=[
            jax.ShapeDtypeStruct((C, D), jnp.float32),
            jax.ShapeDtypeStruct((N, C), jnp.float32),
        ],
        scratch_shapes=[pltpu.VMEM((1, C), jnp.float32)],
        compiler_params=pltpu.CompilerParams(
            dimension_semantics=("arbitrary",),
        ),
    )(features, W, b2)
    return (features_pooled, assignments)


# speedup vs baseline: 1.6189x; 1.6189x over previous
import jax
import jax.numpy as jnp
from jax.experimental import pallas as pl
from jax.experimental.pallas import tpu as pltpu

N = 10000
D = 128
C = 16
BN = 2000
GRID = N // BN

_SELU_ALPHA = 1.6732632423543772848170429916717
_SELU_SCALE = 1.0507009873554804934193349852946


def _dmon_kernel(x_ref, wt_ref, b_ref, pooled_ref, assign_ref, s_ref):
    i = pl.program_id(0)

    x = x_ref[...]
    lt = jax.lax.dot_general(
        wt_ref[...], x,
        (((1,), (1,)), ((), ())),
        preferred_element_type=jnp.float32,
    ) + b_ref[...]

    m = jnp.max(lt, axis=0, keepdims=True)
    e = jnp.exp(lt - m)
    at = e / jnp.sum(e, axis=0, keepdims=True)

    assign_ref[...] = at.T

    part = jax.lax.dot_general(
        at, x,
        (((1,), (0,)), ((), ())),
        preferred_element_type=jnp.float32,
    )
    part_s = jnp.sum(at, axis=1, keepdims=True)

    @pl.when(i == 0)
    def _init():
        pooled_ref[...] = part
        s_ref[...] = part_s

    @pl.when(i > 0)
    def _acc():
        pooled_ref[...] += part
        s_ref[...] += part_s

    @pl.when(i == GRID - 1)
    def _finalize():
        pooled = pooled_ref[...] / s_ref[...]
        pooled_ref[...] = _SELU_SCALE * jnp.where(
            pooled > 0, pooled, _SELU_ALPHA * (jnp.exp(pooled) - 1.0)
        )


def kernel(features, edge_index, W, b):
    del edge_index
    wt = W.T.reshape(C, D)
    b2 = b.reshape(C, 1)
    features_pooled, assignments = pl.pallas_call(
        _dmon_kernel,
        grid=(GRID,),
        in_specs=[
            pl.BlockSpec((BN, D), lambda i: (i, 0)),
            pl.BlockSpec((C, D), lambda i: (0, 0)),
            pl.BlockSpec((C, 1), lambda i: (0, 0)),
        ],
        out_specs=[
            pl.BlockSpec((C, D), lambda i: (0, 0)),
            pl.BlockSpec((BN, C), lambda i: (i, 0)),
        ],
        out_shape=[
            jax.ShapeDtypeStruct((C, D), jnp.float32),
            jax.ShapeDtypeStruct((N, C), jnp.float32),
        ],
        scratch_shapes=[pltpu.VMEM((C, 1), jnp.float32)],
        compiler_params=pltpu.CompilerParams(
            dimension_semantics=("arbitrary",),
        ),
    )(features, wt, b2)
    return (features_pooled, assignments)
